# baseline (speedup 1.0000x reference)
import jax
import jax.numpy as jnp
from jax import lax
from jax.experimental import pallas as pl
from jax.experimental.pallas import tpu as pltpu
from jax.experimental.pallas import tpu_sc as plsc

N_NODES = 10000
N_PAD = 10240
PAD_SRC = 10200
D = 128
HD = 64
WR = HD + 16
E_REAL = 320000 + N_NODES
CHUNK = 256
IDXB = 128
EPW = 21504
E_PAD = EPW * 16
N_CHUNKS = EPW // CHUNK
PAIRS = N_CHUNKS // 2
ROWS_PER_TILE = N_PAD // 16
NI = CHUNK // IDXB


def _vgather(v, idx):
    dn = lax.GatherDimensionNumbers(
        offset_dims=(), collapsed_slice_dims=(0,), start_index_map=(0,))
    return lax.gather(v, idx[:, None], dn, slice_sizes=(1,),
                      mode=lax.GatherScatterMode.PROMISE_IN_BOUNDS)


def _sc_body(q_r, h_r, ats_r, atd_r, zo_r, zs_r, o_out, s_out,
             idx0, idx1, hx0, hx1, ee0, ee1, as0, as1, ad0, ad1,
             o_acc, s_acc, sem_g0, sem_g1, sem_w0, sem_w1):
    c = lax.axis_index("c")
    s = lax.axis_index("s")
    r0 = s * ROWS_PER_TILE

    pltpu.sync_copy(zo_r.at[pl.ds(r0, ROWS_PER_TILE)],
                    o_acc.at[pl.ds(r0, ROWS_PER_TILE)])
    pltpu.sync_copy(zs_r.at[pl.ds(r0, ROWS_PER_TILE)],
                    s_acc.at[pl.ds(r0, ROWS_PER_TILE)])
    plsc.subcore_barrier()

    jvec = [jnp.full((16,), j, jnp.int32) + c * 4 for j in range(4)]
    qbase = s * N_CHUNKS

    def fire(gq, idx_b, hx_b, as_b, ad_b, sem):
        pltpu.sync_copy(q_r.at[gq], idx_b)
        for i in range(NI):
            pltpu.async_copy(h_r.at[c].at[idx_b.at[i]],
                             hx_b.at[pl.ds(i * IDXB, IDXB)], sem)
        for i in range(NI):
            pltpu.async_copy(ats_r.at[idx_b.at[i]],
                             as_b.at[pl.ds(i * IDXB, IDXB)], sem)
        for i in range(NI):
            pltpu.async_copy(atd_r.at[idx_b.at[NI + i]],
                             ad_b.at[pl.ds(i * IDXB, IDXB)], sem)

    def wait_gathers(hx_b, as_b, ad_b, sem):
        for i in range(NI):
            pltpu.make_async_copy(h_r.at[c].at[pl.ds(0, IDXB)],
                                  hx_b.at[pl.ds(i * IDXB, IDXB)], sem).wait()
        for i in range(NI):
            pltpu.make_async_copy(ats_r.at[pl.ds(0, IDXB)],
                                  as_b.at[pl.ds(i * IDXB, IDXB)], sem).wait()
        for i in range(NI):
            pltpu.make_async_copy(atd_r.at[pl.ds(0, IDXB)],
                                  ad_b.at[pl.ds(i * IDXB, IDXB)], sem).wait()

    def compute(hx_b, as_b, ad_b, ee_b):
        @plsc.parallel_loop(0, CHUNK, step=1, unroll=16)
        def _(k):
            t = as_b[k] + ad_b[k]
            ee = jnp.exp(jnp.maximum(t, 0.2 * t))
            ee_b[k] = ee
            for j in range(4):
                m = _vgather(ee, jvec[j])
                hx_b[k, pl.ds(j * 16, 16)] = hx_b[k, pl.ds(j * 16, 16)] * m

    def fire_scatter(idx_b, hs_b, ee_b, sem):
        for i in range(NI):
            pltpu.async_copy(hs_b.at[pl.ds(i * IDXB, IDXB)],
                             o_acc.at[idx_b.at[NI + i]], sem, add=True)

        @pl.when(c == 0)
        def _():
            for i in range(NI):
                pltpu.async_copy(ee_b.at[pl.ds(i * IDXB, IDXB)],
                                 s_acc.at[idx_b.at[NI + i]], sem, add=True)

    def wait_scatter(hs_b, ee_b, sem):
        for i in range(NI):
            pltpu.make_async_copy(hs_b.at[pl.ds(i * IDXB, IDXB)],
                                  o_acc.at[pl.ds(0, IDXB)], sem).wait()

        @pl.when(c == 0)
        def _():
            for i in range(NI):
                pltpu.make_async_copy(ee_b.at[pl.ds(i * IDXB, IDXB)],
                                      s_acc.at[pl.ds(0, IDXB)], sem).wait()

    fire(qbase, idx0, hx0, as0, ad0, sem_g0)

    def pair_body(p, carry):
        g0 = qbase + 2 * p
        fire(g0 + 1, idx1, hx1, as1, ad1, sem_g1)
        wait_gathers(hx0, as0, ad0, sem_g0)
        compute(hx0, as0, ad0, ee0)
        fire_scatter(idx0, hx0, ee0, sem_w0)
        wait_scatter(hx0, ee0, sem_w0)

        @pl.when(p < PAIRS - 1)
        def _():
            fire(g0 + 2, idx0, hx0, as0, ad0, sem_g0)

        wait_gathers(hx1, as1, ad1, sem_g1)
        compute(hx1, as1, ad1, ee1)
        fire_scatter(idx1, hx1, ee1, sem_w1)
        wait_scatter(hx1, ee1, sem_w1)
        return carry

    lax.fori_loop(0, PAIRS, pair_body, 0)
    plsc.subcore_barrier()

    pltpu.sync_copy(o_acc.at[pl.ds(r0, ROWS_PER_TILE)],
                    o_out.at[c].at[pl.ds(r0, ROWS_PER_TILE)])

    @pl.when(c == 0)
    def _():
        pltpu.sync_copy(s_acc.at[pl.ds(r0, ROWS_PER_TILE)],
                        s_out.at[pl.ds(r0, ROWS_PER_TILE)])


def _sc_edge(qarr, h_tab, ats, atd, zo, zs, *, interpret=False):
    mesh = plsc.VectorSubcoreMesh(core_axis_name="c", subcore_axis_name="s",
                                  num_cores=2, num_subcores=16)
    f = pl.kernel(
        _sc_body,
        out_type=(jax.ShapeDtypeStruct((2, N_PAD, HD), jnp.float32),
                  jax.ShapeDtypeStruct((N_PAD, 16), jnp.float32)),
        mesh=mesh,
        scratch_types=[
            pltpu.VMEM((2 * NI, IDXB), jnp.int32),
            pltpu.VMEM((2 * NI, IDXB), jnp.int32),
            pltpu.VMEM((CHUNK, HD), jnp.float32),
            pltpu.VMEM((CHUNK, HD), jnp.float32),
            pltpu.VMEM((CHUNK, 16), jnp.float32),
            pltpu.VMEM((CHUNK, 16), jnp.float32),
            pltpu.VMEM((CHUNK, 16), jnp.float32),
            pltpu.VMEM((CHUNK, 16), jnp.float32),
            pltpu.VMEM((CHUNK, 16), jnp.float32),
            pltpu.VMEM((CHUNK, 16), jnp.float32),
            pltpu.VMEM_SHARED((N_PAD, HD), jnp.float32),
            pltpu.VMEM_SHARED((N_PAD, 16), jnp.float32),
            pltpu.SemaphoreType.DMA,
            pltpu.SemaphoreType.DMA,
            pltpu.SemaphoreType.DMA,
            pltpu.SemaphoreType.DMA,
        ],
        compiler_params=pltpu.CompilerParams(use_tc_tiling_on_sc=False),
        interpret=interpret,
    )
    return f(qarr, h_tab, ats, atd, zo, zs)


_BLK = 2048


def _row_mask():
    rows = jax.lax.broadcasted_iota(jnp.int32, (_BLK, 1), 0)
    return rows + pl.program_id(0) * _BLK < N_NODES


def _tc_head_body(x_ref, w_ref, as_ref, ad_ref, hx_ref, ats_ref, atd_ref):
    h = jnp.dot(x_ref[...], w_ref[...], preferred_element_type=jnp.float32)
    ats = jnp.dot(h, as_ref[...], preferred_element_type=jnp.float32)
    ok = _row_mask()
    atd_ref[...] = jnp.dot(h, ad_ref[...], preferred_element_type=jnp.float32)
    h = jnp.where(ok, h, 0.0)
    hx_ref[0] = h[:, :HD]
    hx_ref[1] = h[:, HD:]
    ats_ref[...] = jnp.where(ok, ats, -1e30)


def _tc_mid_body(o_ref, s_ref, k1_ref, b_ref, w_ref, as_ref, ad_ref,
                 hx_ref, ats_ref, atd_ref):
    p = jnp.concatenate([o_ref[0], o_ref[1]], axis=1)
    rep = jnp.dot(s_ref[...], k1_ref[...], preferred_element_type=jnp.float32)
    h = p / (rep + 1e-16) + b_ref[...]
    h = jnp.where(h > 0, h, jnp.exp(h) - 1.0)
    h2 = jnp.dot(h, w_ref[...], preferred_element_type=jnp.float32)
    ats = jnp.dot(h2, as_ref[...], preferred_element_type=jnp.float32)
    ok = _row_mask()
    atd_ref[...] = jnp.dot(h2, ad_ref[...], preferred_element_type=jnp.float32)
    h2 = jnp.where(ok, h2, 0.0)
    hx_ref[0] = h2[:, :HD]
    hx_ref[1] = h2[:, HD:]
    ats_ref[...] = jnp.where(ok, ats, -1e30)


def _tc_fin_body(o_ref, s_ref, k2_ref, b_ref, out_ref):
    p = jnp.concatenate([o_ref[0], o_ref[1]], axis=1)
    rep = jnp.dot(s_ref[...], k2_ref[...], preferred_element_type=jnp.float32)
    out_ref[...] = p / (rep + 1e-16) + b_ref[...]


def _tc_head(xp, W, As, Ad, *, interpret=False):
    return pl.pallas_call(
        _tc_head_body,
        grid=(N_PAD // _BLK,),
        in_specs=[pl.BlockSpec((_BLK, 128), lambda i: (i, 0)),
                  pl.BlockSpec((128, 128), lambda i: (0, 0)),
                  pl.BlockSpec((128, 16), lambda i: (0, 0)),
                  pl.BlockSpec((128, 16), lambda i: (0, 0))],
        out_specs=[pl.BlockSpec((2, _BLK, HD), lambda i: (0, i, 0)),
                   pl.BlockSpec((_BLK, 16), lambda i: (i, 0)),
                   pl.BlockSpec((_BLK, 16), lambda i: (i, 0))],
        out_shape=[jax.ShapeDtypeStruct((2, N_PAD, HD), jnp.float32),
                   jax.ShapeDtypeStruct((N_PAD, 16), jnp.float32),
                   jax.ShapeDtypeStruct((N_PAD, 16), jnp.float32)],
        interpret=interpret,
    )(xp, W, As, Ad)


def _tc_mid(o1, s1, K1, b1, W2, As2, Ad2, *, interpret=False):
    return pl.pallas_call(
        _tc_mid_body,
        grid=(N_PAD // _BLK,),
        in_specs=[pl.BlockSpec((2, _BLK, HD), lambda i: (0, i, 0)),
                  pl.BlockSpec((_BLK, 16), lambda i: (i, 0)),
                  pl.BlockSpec((16, 128), lambda i: (0, 0)),
                  pl.BlockSpec((1, 128), lambda i: (0, 0)),
                  pl.BlockSpec((128, 128), lambda i: (0, 0)),
                  pl.BlockSpec((128, 16), lambda i: (0, 0)),
                  pl.BlockSpec((128, 16), lambda i: (0, 0))],
        out_specs=[pl.BlockSpec((2, _BLK, HD), lambda i: (0, i, 0)),
                   pl.BlockSpec((_BLK, 16), lambda i: (i, 0)),
                   pl.BlockSpec((_BLK, 16), lambda i: (i, 0))],
        out_shape=[jax.ShapeDtypeStruct((2, N_PAD, HD), jnp.float32),
                   jax.ShapeDtypeStruct((N_PAD, 16), jnp.float32),
                   jax.ShapeDtypeStruct((N_PAD, 16), jnp.float32)],
        interpret=interpret,
    )(o1, s1, K1, b1, W2, As2, Ad2)


def _tc_fin(o2, s2, K2, b2, *, interpret=False):
    return pl.pallas_call(
        _tc_fin_body,
        grid=(N_PAD // _BLK,),
        in_specs=[pl.BlockSpec((2, _BLK, HD), lambda i: (0, i, 0)),
                  pl.BlockSpec((_BLK, 16), lambda i: (i, 0)),
                  pl.BlockSpec((16, 128), lambda i: (0, 0)),
                  pl.BlockSpec((1, 128), lambda i: (0, 0))],
        out_specs=pl.BlockSpec((_BLK, 128), lambda i: (i, 0)),
        out_shape=jax.ShapeDtypeStruct((N_PAD, 128), jnp.float32),
        interpret=interpret,
    )(o2, s2, K2, b2)


def _prep(x, edge_index, att_src1, att_dst1, att_src2, att_dst2):
    loops = jnp.arange(N_NODES, dtype=edge_index.dtype)
    src = jnp.concatenate([edge_index[0], loops])
    dst = jnp.concatenate([edge_index[1], loops])
    pad_s = jnp.full((E_PAD - E_REAL,), PAD_SRC, dtype=src.dtype)
    pad_d = jnp.zeros((E_PAD - E_REAL,), dtype=dst.dtype)
    src4 = jnp.concatenate([src, pad_s]).reshape(16, N_CHUNKS, NI,
                                                 IDXB).astype(jnp.int32)
    dst4 = jnp.concatenate([dst, pad_d]).reshape(16, N_CHUNKS, NI,
                                                 IDXB).astype(jnp.int32)
    qarr = jnp.concatenate([src4, dst4], axis=2).reshape(
        16 * N_CHUNKS, 2 * NI, IDXB)
    xp = jnp.concatenate(
        [x, jnp.zeros((N_PAD - N_NODES, D), jnp.float32)], axis=0)

    eye8 = jnp.eye(8, dtype=jnp.float32)
    z816 = jnp.zeros((128, 8), jnp.float32)
    a1s = (att_src1[0][:, :, None] * eye8[:, None, :]).reshape(128, 8)
    a1d = (att_dst1[0][:, :, None] * eye8[:, None, :]).reshape(128, 8)
    As1 = jnp.concatenate([a1s, z816], axis=1)
    Ad1 = jnp.concatenate([a1d, z816], axis=1)
    As2 = jnp.concatenate(
        [jnp.broadcast_to(att_src2[0, 0][:, None], (128, 8)), z816], axis=1)
    Ad2 = jnp.concatenate(
        [jnp.broadcast_to(att_dst2[0, 0][:, None], (128, 8)), z816], axis=1)
    K1 = jnp.concatenate([jnp.repeat(eye8, 16, axis=1),
                          jnp.zeros((8, 128), jnp.float32)], axis=0)
    K2 = jnp.concatenate([jnp.full((8, 128), 0.125, jnp.float32),
                          jnp.zeros((8, 128), jnp.float32)], axis=0)
    zo = jnp.zeros((N_PAD, HD), jnp.float32)
    zs = jnp.zeros((N_PAD, 16), jnp.float32)
    return qarr, xp, As1, Ad1, As2, Ad2, K1, K2, zo, zs


def _gat2(x, edge_index, W1, att_src1, att_dst1, b1, W2, att_src2, att_dst2,
          b2, interpret=False):
    qarr, xp, As1, Ad1, As2, Ad2, K1, K2, zo, zs = _prep(
        x, edge_index, att_src1, att_dst1, att_src2, att_dst2)
    h1, ats1, atd1 = _tc_head(xp, W1, As1, Ad1, interpret=interpret)
    o1, s1 = _sc_edge(qarr, h1, ats1, atd1, zo, zs, interpret=interpret)
    h2, ats2, atd2 = _tc_mid(o1, s1, K1, b1.reshape(1, 128), W2, As2, Ad2,
                             interpret=interpret)
    o2, s2 = _sc_edge(qarr, h2, ats2, atd2, zo, zs, interpret=interpret)
    out = _tc_fin(o2, s2, K2, b2.reshape(1, 128), interpret=interpret)
    return out[:N_NODES]


def kernel(x, edge_index, W1, att_src1, att_dst1, b1, W2, att_src2, att_dst2,
           b2):
    return _gat2(x, edge_index, W1, att_src1, att_dst1, b1, W2, att_src2,
                 att_dst2, b2)


# speedup vs baseline: 1.0454x; 1.0454x over previous
import jax
import jax.numpy as jnp
from jax import lax
from jax.experimental import pallas as pl
from jax.experimental.pallas import tpu as pltpu
from jax.experimental.pallas import tpu_sc as plsc

N_NODES = 10000
N_PAD = 10240
PAD_SRC = 10200
D = 128
HD = 64
WR = HD + 16
E_REAL = 320000 + N_NODES
CHUNK = 256
IDXB = 128
EPW = 21504
E_PAD = EPW * 16
N_CHUNKS = EPW // CHUNK
PAIRS = N_CHUNKS // 2
ROWS_PER_TILE = N_PAD // 16
NI = CHUNK // IDXB


def _vgather(v, idx):
    dn = lax.GatherDimensionNumbers(
        offset_dims=(), collapsed_slice_dims=(0,), start_index_map=(0,))
    return lax.gather(v, idx[:, None], dn, slice_sizes=(1,),
                      mode=lax.GatherScatterMode.PROMISE_IN_BOUNDS)


def _sc_body(q_r, h_r, ats_r, atd_r, zo_r, zs_r, o_out, s_out,
             idx0, idx1, hx0, hx1, ee0, ee1, as0, as1, ad0, ad1,
             o_acc, s_acc, sem_g0, sem_g1, sem_w0, sem_w1):
    c = lax.axis_index("c")
    s = lax.axis_index("s")
    r0 = s * ROWS_PER_TILE

    pltpu.sync_copy(zo_r.at[pl.ds(r0, ROWS_PER_TILE)],
                    o_acc.at[pl.ds(r0, ROWS_PER_TILE)])
    pltpu.sync_copy(zs_r.at[pl.ds(r0, ROWS_PER_TILE)],
                    s_acc.at[pl.ds(r0, ROWS_PER_TILE)])
    plsc.subcore_barrier()

    jvec = [jnp.full((16,), j, jnp.int32) + c * 4 for j in range(4)]
    qbase = s * N_CHUNKS

    def fire(gq, idx_b, hx_b, as_b, ad_b, sem):
        pltpu.sync_copy(q_r.at[gq], idx_b)
        for i in range(NI):
            pltpu.async_copy(h_r.at[c].at[idx_b.at[i]],
                             hx_b.at[pl.ds(i * IDXB, IDXB)], sem)
        for i in range(NI):
            pltpu.async_copy(ats_r.at[idx_b.at[i]],
                             as_b.at[pl.ds(i * IDXB, IDXB)], sem)
        for i in range(NI):
            pltpu.async_copy(atd_r.at[idx_b.at[NI + i]],
                             ad_b.at[pl.ds(i * IDXB, IDXB)], sem)

    def wait_gathers(hx_b, as_b, ad_b, sem):
        for i in range(NI):
            pltpu.make_async_copy(h_r.at[c].at[pl.ds(0, IDXB)],
                                  hx_b.at[pl.ds(i * IDXB, IDXB)], sem).wait()
        for i in range(NI):
            pltpu.make_async_copy(ats_r.at[pl.ds(0, IDXB)],
                                  as_b.at[pl.ds(i * IDXB, IDXB)], sem).wait()
        for i in range(NI):
            pltpu.make_async_copy(atd_r.at[pl.ds(0, IDXB)],
                                  ad_b.at[pl.ds(i * IDXB, IDXB)], sem).wait()

    def compute(hx_b, as_b, ad_b, ee_b):
        @plsc.parallel_loop(0, CHUNK, step=1, unroll=8)
        def _(k):
            t = as_b[k] + ad_b[k]
            ee = jnp.exp(jnp.maximum(t, 0.2 * t))
            ee_b[k] = ee
            for j in range(4):
                m = _vgather(ee, jvec[j])
                hx_b[k, pl.ds(j * 16, 16)] = hx_b[k, pl.ds(j * 16, 16)] * m

    def fire_scatter(idx_b, hs_b, ee_b, sem):
        for i in range(NI):
            pltpu.async_copy(hs_b.at[pl.ds(i * IDXB, IDXB)],
                             o_acc.at[idx_b.at[NI + i]], sem, add=True)

        @pl.when(c == 0)
        def _():
            for i in range(NI):
                pltpu.async_copy(ee_b.at[pl.ds(i * IDXB, IDXB)],
                                 s_acc.at[idx_b.at[NI + i]], sem, add=True)

    def wait_scatter(hs_b, ee_b, sem):
        for i in range(NI):
            pltpu.make_async_copy(hs_b.at[pl.ds(i * IDXB, IDXB)],
                                  o_acc.at[pl.ds(0, IDXB)], sem).wait()

        @pl.when(c == 0)
        def _():
            for i in range(NI):
                pltpu.make_async_copy(ee_b.at[pl.ds(i * IDXB, IDXB)],
                                      s_acc.at[pl.ds(0, IDXB)], sem).wait()

    fire(qbase, idx0, hx0, as0, ad0, sem_g0)

    def pair_body(p, carry):
        g0 = qbase + 2 * p
        fire(g0 + 1, idx1, hx1, as1, ad1, sem_g1)
        wait_gathers(hx0, as0, ad0, sem_g0)
        compute(hx0, as0, ad0, ee0)
        fire_scatter(idx0, hx0, ee0, sem_w0)
        wait_scatter(hx0, ee0, sem_w0)

        @pl.when(p < PAIRS - 1)
        def _():
            fire(g0 + 2, idx0, hx0, as0, ad0, sem_g0)

        wait_gathers(hx1, as1, ad1, sem_g1)
        compute(hx1, as1, ad1, ee1)
        fire_scatter(idx1, hx1, ee1, sem_w1)
        wait_scatter(hx1, ee1, sem_w1)
        return carry

    lax.fori_loop(0, PAIRS, pair_body, 0)
    plsc.subcore_barrier()

    pltpu.sync_copy(o_acc.at[pl.ds(r0, ROWS_PER_TILE)],
                    o_out.at[c].at[pl.ds(r0, ROWS_PER_TILE)])

    @pl.when(c == 0)
    def _():
        pltpu.sync_copy(s_acc.at[pl.ds(r0, ROWS_PER_TILE)],
                        s_out.at[pl.ds(r0, ROWS_PER_TILE)])


def _sc_edge(qarr, h_tab, ats, atd, zo, zs, *, interpret=False):
    mesh = plsc.VectorSubcoreMesh(core_axis_name="c", subcore_axis_name="s",
                                  num_cores=2, num_subcores=16)
    f = pl.kernel(
        _sc_body,
        out_type=(jax.ShapeDtypeStruct((2, N_PAD, HD), jnp.float32),
                  jax.ShapeDtypeStruct((N_PAD, 16), jnp.float32)),
        mesh=mesh,
        scratch_types=[
            pltpu.VMEM((2 * NI, IDXB), jnp.int32),
            pltpu.VMEM((2 * NI, IDXB), jnp.int32),
            pltpu.VMEM((CHUNK, HD), jnp.float32),
            pltpu.VMEM((CHUNK, HD), jnp.float32),
            pltpu.VMEM((CHUNK, 16), jnp.float32),
            pltpu.VMEM((CHUNK, 16), jnp.float32),
            pltpu.VMEM((CHUNK, 16), jnp.float32),
            pltpu.VMEM((CHUNK, 16), jnp.float32),
            pltpu.VMEM((CHUNK, 16), jnp.float32),
            pltpu.VMEM((CHUNK, 16), jnp.float32),
            pltpu.VMEM_SHARED((N_PAD, HD), jnp.float32),
            pltpu.VMEM_SHARED((N_PAD, 16), jnp.float32),
            pltpu.SemaphoreType.DMA,
            pltpu.SemaphoreType.DMA,
            pltpu.SemaphoreType.DMA,
            pltpu.SemaphoreType.DMA,
        ],
        compiler_params=pltpu.CompilerParams(use_tc_tiling_on_sc=False),
        interpret=interpret,
    )
    return f(qarr, h_tab, ats, atd, zo, zs)


_BLK = 2048


def _row_mask():
    rows = jax.lax.broadcasted_iota(jnp.int32, (_BLK, 1), 0)
    return rows + pl.program_id(0) * _BLK < N_NODES


def _tc_head_body(x_ref, w_ref, as_ref, ad_ref, hx_ref, ats_ref, atd_ref):
    h = jnp.dot(x_ref[...], w_ref[...], preferred_element_type=jnp.float32)
    ats = jnp.dot(h, as_ref[...], preferred_element_type=jnp.float32)
    ok = _row_mask()
    atd_ref[...] = jnp.dot(h, ad_ref[...], preferred_element_type=jnp.float32)
    h = jnp.where(ok, h, 0.0)
    hx_ref[0] = h[:, :HD]
    hx_ref[1] = h[:, HD:]
    ats_ref[...] = jnp.where(ok, ats, -1e30)


def _tc_mid_body(o_ref, s_ref, k1_ref, b_ref, w_ref, as_ref, ad_ref,
                 hx_ref, ats_ref, atd_ref):
    p = jnp.concatenate([o_ref[0], o_ref[1]], axis=1)
    rep = jnp.dot(s_ref[...], k1_ref[...], preferred_element_type=jnp.float32)
    h = p / (rep + 1e-16) + b_ref[...]
    h = jnp.where(h > 0, h, jnp.exp(h) - 1.0)
    h2 = jnp.dot(h, w_ref[...], preferred_element_type=jnp.float32)
    ats = jnp.dot(h2, as_ref[...], preferred_element_type=jnp.float32)
    ok = _row_mask()
    atd_ref[...] = jnp.dot(h2, ad_ref[...], preferred_element_type=jnp.float32)
    h2 = jnp.where(ok, h2, 0.0)
    hx_ref[0] = h2[:, :HD]
    hx_ref[1] = h2[:, HD:]
    ats_ref[...] = jnp.where(ok, ats, -1e30)


def _tc_fin_body(o_ref, s_ref, k2_ref, b_ref, out_ref):
    p = jnp.concatenate([o_ref[0], o_ref[1]], axis=1)
    rep = jnp.dot(s_ref[...], k2_ref[...], preferred_element_type=jnp.float32)
    out_ref[...] = p / (rep + 1e-16) + b_ref[...]


def _tc_head(xp, W, As, Ad, *, interpret=False):
    return pl.pallas_call(
        _tc_head_body,
        grid=(N_PAD // _BLK,),
        in_specs=[pl.BlockSpec((_BLK, 128), lambda i: (i, 0)),
                  pl.BlockSpec((128, 128), lambda i: (0, 0)),
                  pl.BlockSpec((128, 16), lambda i: (0, 0)),
                  pl.BlockSpec((128, 16), lambda i: (0, 0))],
        out_specs=[pl.BlockSpec((2, _BLK, HD), lambda i: (0, i, 0)),
                   pl.BlockSpec((_BLK, 16), lambda i: (i, 0)),
                   pl.BlockSpec((_BLK, 16), lambda i: (i, 0))],
        out_shape=[jax.ShapeDtypeStruct((2, N_PAD, HD), jnp.float32),
                   jax.ShapeDtypeStruct((N_PAD, 16), jnp.float32),
                   jax.ShapeDtypeStruct((N_PAD, 16), jnp.float32)],
        interpret=interpret,
    )(xp, W, As, Ad)


def _tc_mid(o1, s1, K1, b1, W2, As2, Ad2, *, interpret=False):
    return pl.pallas_call(
        _tc_mid_body,
        grid=(N_PAD // _BLK,),
        in_specs=[pl.BlockSpec((2, _BLK, HD), lambda i: (0, i, 0)),
                  pl.BlockSpec((_BLK, 16), lambda i: (i, 0)),
                  pl.BlockSpec((16, 128), lambda i: (0, 0)),
                  pl.BlockSpec((1, 128), lambda i: (0, 0)),
                  pl.BlockSpec((128, 128), lambda i: (0, 0)),
                  pl.BlockSpec((128, 16), lambda i: (0, 0)),
                  pl.BlockSpec((128, 16), lambda i: (0, 0))],
        out_specs=[pl.BlockSpec((2, _BLK, HD), lambda i: (0, i, 0)),
                   pl.BlockSpec((_BLK, 16), lambda i: (i, 0)),
                   pl.BlockSpec((_BLK, 16), lambda i: (i, 0))],
        out_shape=[jax.ShapeDtypeStruct((2, N_PAD, HD), jnp.float32),
                   jax.ShapeDtypeStruct((N_PAD, 16), jnp.float32),
                   jax.ShapeDtypeStruct((N_PAD, 16), jnp.float32)],
        interpret=interpret,
    )(o1, s1, K1, b1, W2, As2, Ad2)


def _tc_fin(o2, s2, K2, b2, *, interpret=False):
    return pl.pallas_call(
        _tc_fin_body,
        grid=(N_PAD // _BLK,),
        in_specs=[pl.BlockSpec((2, _BLK, HD), lambda i: (0, i, 0)),
                  pl.BlockSpec((_BLK, 16), lambda i: (i, 0)),
                  pl.BlockSpec((16, 128), lambda i: (0, 0)),
                  pl.BlockSpec((1, 128), lambda i: (0, 0))],
        out_specs=pl.BlockSpec((_BLK, 128), lambda i: (i, 0)),
        out_shape=jax.ShapeDtypeStruct((N_PAD, 128), jnp.float32),
        interpret=interpret,
    )(o2, s2, K2, b2)


def _prep(x, edge_index, att_src1, att_dst1, att_src2, att_dst2):
    loops = jnp.arange(N_NODES, dtype=edge_index.dtype)
    src = jnp.concatenate([edge_index[0], loops])
    dst = jnp.concatenate([edge_index[1], loops])
    pad_s = jnp.full((E_PAD - E_REAL,), PAD_SRC, dtype=src.dtype)
    pad_d = jnp.zeros((E_PAD - E_REAL,), dtype=dst.dtype)
    src4 = jnp.concatenate([src, pad_s]).reshape(16, N_CHUNKS, NI,
                                                 IDXB).astype(jnp.int32)
    dst4 = jnp.concatenate([dst, pad_d]).reshape(16, N_CHUNKS, NI,
                                                 IDXB).astype(jnp.int32)
    qarr = jnp.concatenate([src4, dst4], axis=2).reshape(
        16 * N_CHUNKS, 2 * NI, IDXB)
    xp = jnp.concatenate(
        [x, jnp.zeros((N_PAD - N_NODES, D), jnp.float32)], axis=0)

    eye8 = jnp.eye(8, dtype=jnp.float32)
    z816 = jnp.zeros((128, 8), jnp.float32)
    a1s = (att_src1[0][:, :, None] * eye8[:, None, :]).reshape(128, 8)
    a1d = (att_dst1[0][:, :, None] * eye8[:, None, :]).reshape(128, 8)
    As1 = jnp.concatenate([a1s, z816], axis=1)
    Ad1 = jnp.concatenate([a1d, z816], axis=1)
    As2 = jnp.concatenate(
        [jnp.broadcast_to(att_src2[0, 0][:, None], (128, 8)), z816], axis=1)
    Ad2 = jnp.concatenate(
        [jnp.broadcast_to(att_dst2[0, 0][:, None], (128, 8)), z816], axis=1)
    K1 = jnp.concatenate([jnp.repeat(eye8, 16, axis=1),
                          jnp.zeros((8, 128), jnp.float32)], axis=0)
    K2 = jnp.concatenate([jnp.full((8, 128), 0.125, jnp.float32),
                          jnp.zeros((8, 128), jnp.float32)], axis=0)
    zo = jnp.zeros((N_PAD, HD), jnp.float32)
    zs = jnp.zeros((N_PAD, 16), jnp.float32)
    return qarr, xp, As1, Ad1, As2, Ad2, K1, K2, zo, zs


def _gat2(x, edge_index, W1, att_src1, att_dst1, b1, W2, att_src2, att_dst2,
          b2, interpret=False):
    qarr, xp, As1, Ad1, As2, Ad2, K1, K2, zo, zs = _prep(
        x, edge_index, att_src1, att_dst1, att_src2, att_dst2)
    h1, ats1, atd1 = _tc_head(xp, W1, As1, Ad1, interpret=interpret)
    o1, s1 = _sc_edge(qarr, h1, ats1, atd1, zo, zs, interpret=interpret)
    h2, ats2, atd2 = _tc_mid(o1, s1, K1, b1.reshape(1, 128), W2, As2, Ad2,
                             interpret=interpret)
    o2, s2 = _sc_edge(qarr, h2, ats2, atd2, zo, zs, interpret=interpret)
    out = _tc_fin(o2, s2, K2, b2.reshape(1, 128), interpret=interpret)
    return out[:N_NODES]


def kernel(x, edge_index, W1, att_src1, att_dst1, b1, W2, att_src2, att_dst2,
           b2):
    return _gat2(x, edge_index, W1, att_src1, att_dst1, b1, W2, att_src2,
                 att_dst2, b2)
